# Initial kernel scaffold; baseline (speedup 1.0000x reference)
import jax
import jax.numpy as jnp
from jax.experimental import pallas as pl


def kernel(rpn_cls, rpn_reg, anchors):
    raise NotImplementedError("write your pallas kernel here")


# speedup vs baseline: 10.8762x; 10.8762x over previous
import functools

import jax
import jax.numpy as jnp
from jax.experimental import pallas as pl
from jax.experimental.pallas import tpu as pltpu

_A = 20000
_TOP_N = 200
_IOU_THR = 0.7
_LANES = 128

_NEG_INF = float("-inf")


def _nms_body(a_valid, top_n, iou_thr, cls_ref, reg_ref, anch_ref, out_ref,
              by0, bx0, by1, bx1, barea):
    rows = cls_ref.shape[1]
    row_iota = jax.lax.broadcasted_iota(jnp.int32, (rows, _LANES), 0)
    col_iota = jax.lax.broadcasted_iota(jnp.int32, (rows, _LANES), 1)
    flat_iota = row_iota * _LANES + col_iota
    valid = flat_iota < a_valid

    scores = jax.nn.sigmoid(cls_ref[0])
    scores = jnp.where(valid, scores, _NEG_INF)

    aymin = anch_ref[0]
    axmin = anch_ref[1]
    aymax = anch_ref[2]
    axmax = anch_ref[3]
    ah = aymax - aymin
    aw = axmax - axmin
    acy = aymin + 0.5 * ah
    acx = axmin + 0.5 * aw
    dy = reg_ref[0, 0]
    dx = reg_ref[0, 1]
    dh = reg_ref[0, 2]
    dw = reg_ref[0, 3]
    pcy = dy * ah + acy
    pcx = dx * aw + acx
    ph = jnp.exp(dh) * ah
    pw = jnp.exp(dw) * aw
    y0 = jnp.clip(pcy - 0.5 * ph, 0.0, 1.0)
    x0 = jnp.clip(pcx - 0.5 * pw, 0.0, 1.0)
    y1 = jnp.clip(pcy + 0.5 * ph, 0.0, 1.0)
    x1 = jnp.clip(pcx + 0.5 * pw, 0.0, 1.0)
    by0[...] = y0
    bx0[...] = x0
    by1[...] = y1
    bx1[...] = x1
    barea[...] = (y1 - y0) * (x1 - x0)

    sel_iota = jax.lax.broadcasted_iota(jnp.int32, (8, top_n), 1)

    def body(i, state):
        s, sel = state
        m = jnp.max(s)
        idx = jnp.min(jnp.where(s == m, flat_iota, rows * _LANES))
        r = idx // _LANES
        c = idx % _LANES
        cmask = col_iota[0:1, :] == c

        def pick(ref):
            rowv = ref[pl.ds(r, 1), :]
            return jnp.sum(jnp.where(cmask, rowv, 0.0))

        sy0 = pick(by0)
        sx0 = pick(bx0)
        sy1 = pick(by1)
        sx1 = pick(bx1)
        sarea = (sy1 - sy0) * (sx1 - sx0)

        yi0 = jnp.maximum(sy0, by0[...])
        xi0 = jnp.maximum(sx0, bx0[...])
        yi1 = jnp.minimum(sy1, by1[...])
        xi1 = jnp.minimum(sx1, bx1[...])
        inter = jnp.maximum(yi1 - yi0, 0.0) * jnp.maximum(xi1 - xi0, 0.0)
        iou = inter / (sarea + barea[...] - inter + 1e-8)

        s = jnp.where((iou > iou_thr) | (flat_iota == idx), _NEG_INF, s)

        comp = jnp.stack([sy0, sx0, sy1, sx1, sy0, sx0, sy1, sx1])
        sel = jnp.where(sel_iota == i, comp[:, None], sel)
        return (s, sel)

    sel0 = jnp.zeros((8, top_n), jnp.float32)
    _, sel = jax.lax.fori_loop(0, top_n, body, (scores, sel0))
    out_ref[0] = sel


def _proposal_tc(rpn_cls, rpn_reg, anchors, a_valid, top_n, iou_thr):
    b = rpn_cls.shape[0]
    a = rpn_cls.shape[1]
    a_pad = ((a + _LANES * 8 - 1) // (_LANES * 8)) * (_LANES * 8)
    rows = a_pad // _LANES

    cls_p = jnp.pad(rpn_cls[..., 0], ((0, 0), (0, a_pad - a))).reshape(
        b, rows, _LANES)
    reg_p = jnp.pad(jnp.transpose(rpn_reg, (0, 2, 1)),
                    ((0, 0), (0, 0), (0, a_pad - a))).reshape(b, 4, rows, _LANES)
    anch_p = jnp.pad(jnp.transpose(anchors, (1, 0)),
                     ((0, 0), (0, a_pad - a))).reshape(4, rows, _LANES)

    kern = functools.partial(_nms_body, a_valid, top_n, iou_thr)
    out = pl.pallas_call(
        kern,
        grid=(b,),
        in_specs=[
            pl.BlockSpec((1, rows, _LANES), lambda i: (i, 0, 0)),
            pl.BlockSpec((1, 4, rows, _LANES), lambda i: (i, 0, 0, 0)),
            pl.BlockSpec((4, rows, _LANES), lambda i: (0, 0, 0)),
        ],
        out_specs=pl.BlockSpec((1, 8, top_n), lambda i: (i, 0, 0)),
        out_shape=jax.ShapeDtypeStruct((b, 8, top_n), jnp.float32),
        scratch_shapes=[pltpu.VMEM((rows, _LANES), jnp.float32)] * 5,
    )(cls_p, reg_p, anch_p)
    proposals = jnp.transpose(out[:, :4, :], (0, 2, 1)).reshape(b * top_n, 4)
    indices = jnp.zeros((b * top_n,), jnp.int32)
    return proposals, indices


def kernel(rpn_cls, rpn_reg, anchors):
    return _proposal_tc(rpn_cls, rpn_reg, anchors, _A, _TOP_N, _IOU_THR)


# speedup vs baseline: 100.6729x; 9.2563x over previous
import functools

import jax
import jax.numpy as jnp
from jax import lax
from jax.experimental import pallas as pl
from jax.experimental.pallas import tpu as pltpu
from jax.experimental.pallas import tpu_sc as plsc

_A = 20000
_B = 8
_TOP_N = 200
_IOU_THR = 0.7
_LANES = 128

_A_PAD = 20480
_ROWS = _A_PAD // _LANES
_KCAP = 208
_L1N = _A_PAD // 16
_L2N = _L1N // 16
_BIG = 1 << 30
_NEG_INF = float("-inf")


def _decode_body(a_valid, cls_ref, reg_ref, anch_ref,
                 sc_ref, y0_ref, x0_ref, y1_ref, x1_ref):
    row_iota = lax.broadcasted_iota(jnp.int32, (_ROWS, _LANES), 0)
    col_iota = lax.broadcasted_iota(jnp.int32, (_ROWS, _LANES), 1)
    valid = (row_iota * _LANES + col_iota) < a_valid

    scores = jax.nn.sigmoid(cls_ref[0])
    sc_ref[0] = jnp.where(valid, scores, _NEG_INF)

    aymin = anch_ref[0]
    axmin = anch_ref[1]
    aymax = anch_ref[2]
    axmax = anch_ref[3]
    ah = aymax - aymin
    aw = axmax - axmin
    acy = aymin + 0.5 * ah
    acx = axmin + 0.5 * aw
    dy = reg_ref[0, 0]
    dx = reg_ref[0, 1]
    dh = reg_ref[0, 2]
    dw = reg_ref[0, 3]
    pcy = dy * ah + acy
    pcx = dx * aw + acx
    ph = jnp.exp(dh) * ah
    pw = jnp.exp(dw) * aw
    y0_ref[0] = jnp.clip(pcy - 0.5 * ph, 0.0, 1.0)
    x0_ref[0] = jnp.clip(pcx - 0.5 * pw, 0.0, 1.0)
    y1_ref[0] = jnp.clip(pcy + 0.5 * ph, 0.0, 1.0)
    x1_ref[0] = jnp.clip(pcx + 0.5 * pw, 0.0, 1.0)


def _decode_tc(rpn_cls, rpn_reg, anchors, b, a_valid, interpret=False):
    pad = _A_PAD - a_valid
    cls_p = jnp.pad(rpn_cls[..., 0], ((0, 0), (0, pad))).reshape(
        b, _ROWS, _LANES)
    reg_p = jnp.pad(jnp.transpose(rpn_reg, (0, 2, 1)),
                    ((0, 0), (0, 0), (0, pad))).reshape(b, 4, _ROWS, _LANES)
    anch_p = jnp.pad(jnp.transpose(anchors, (1, 0)),
                     ((0, 0), (0, pad))).reshape(4, _ROWS, _LANES)

    plane = jax.ShapeDtypeStruct((b, _ROWS, _LANES), jnp.float32)
    outs = pl.pallas_call(
        functools.partial(_decode_body, a_valid),
        grid=(b,),
        in_specs=[
            pl.BlockSpec((1, _ROWS, _LANES), lambda i: (i, 0, 0)),
            pl.BlockSpec((1, 4, _ROWS, _LANES), lambda i: (i, 0, 0, 0)),
            pl.BlockSpec((4, _ROWS, _LANES), lambda i: (0, 0, 0)),
        ],
        out_specs=[pl.BlockSpec((1, _ROWS, _LANES), lambda i: (i, 0, 0))] * 5,
        out_shape=[plane] * 5,
        interpret=interpret,
    )(cls_p, reg_p, anch_p)
    return [o.reshape(b, _A_PAD) for o in outs]


def _axis_ids():
    return lax.axis_index("c"), lax.axis_index("s")


def _store1(ref, pos, val):
    plsc.store_scatter(ref, [jnp.full((16,), pos, jnp.int32)],
                       jnp.full((16,), val, jnp.float32),
                       mask=lax.iota(jnp.int32, 16) == 0)


def _sc_nms_body(top_n, iou_thr, n_samples,
                 sc_hbm, y0_hbm, x0_hbm, y1_hbm, x1_hbm, out_hbm,
                 sc_v, y0_v, x0_v, y1_v, x1_v, l1_v, l2_v,
                 ky0_v, kx0_v, ky1_v, kx1_v, kar_v):
    c, s = _axis_ids()
    n_cores = 2
    per_core = n_samples // n_cores

    @pl.when(s < per_core)
    def _work():
        samp = c * per_core + s
        pltpu.sync_copy(sc_hbm.at[samp], sc_v)
        pltpu.sync_copy(y0_hbm.at[samp], y0_v)
        pltpu.sync_copy(x0_hbm.at[samp], x0_v)
        pltpu.sync_copy(y1_hbm.at[samp], y1_v)
        pltpu.sync_copy(x1_hbm.at[samp], x1_v)

        iota = lax.iota(jnp.int32, 16)
        zeros16 = jnp.zeros((16,), jnp.float32)

        for j in range(_KCAP // 16):
            ky0_v[pl.ds(j * 16, 16)] = zeros16
            kx0_v[pl.ds(j * 16, 16)] = zeros16
            ky1_v[pl.ds(j * 16, 16)] = zeros16
            kx1_v[pl.ds(j * 16, 16)] = zeros16
            kar_v[pl.ds(j * 16, 16)] = zeros16

        def l1_build(i, _):
            ch = sc_v[pl.ds(i * 16, 16)]
            _store1(l1_v, i, jnp.max(ch))
            return 0
        lax.fori_loop(0, _L1N, l1_build, 0)

        def l2_build(i, _):
            ch = l1_v[pl.ds(i * 16, 16)]
            _store1(l2_v, i, jnp.max(ch))
            return 0
        lax.fori_loop(0, _L2N, l2_build, 0)

        def wcond(state):
            kn, alive = state
            return (kn < top_n) & (alive > 0)

        def wbody(state):
            kn, alive = state

            l2chunks = [l2_v[pl.ds(k * 16, 16)] for k in range(_L2N // 16)]
            vmax = l2chunks[0]
            for ch in l2chunks[1:]:
                vmax = jnp.maximum(vmax, ch)
            m = jnp.max(vmax)

            p2 = _BIG
            for k, ch in enumerate(l2chunks):
                cand = jnp.where(ch == m, k * 16 + iota, _BIG)
                p2 = jnp.minimum(p2, jnp.min(cand))

            idx1 = p2 * 16 + iota
            ch1 = l1_v[pl.ds(p2 * 16, 16)]
            p1 = jnp.min(jnp.where(ch1 == m, idx1, _BIG))

            idx0 = p1 * 16 + iota
            ch0 = sc_v[pl.ds(p1 * 16, 16)]
            p = jnp.min(jnp.where(ch0 == m, idx0, _BIG))

            live = m > _NEG_INF

            @pl.when(live)
            def _consume():
                ch0n = jnp.where(idx0 == p, _NEG_INF, ch0)
                sc_v[pl.ds(p1 * 16, 16)] = ch0n
                m1 = jnp.max(ch0n)
                _store1(l1_v, p1, m1)
                ch1n = jnp.where(idx1 == p1, m1, ch1)
                _store1(l2_v, p2, jnp.max(ch1n))

            hit = idx0 == p

            def pick(ref):
                v = ref[pl.ds(p1 * 16, 16)]
                return jnp.sum(jnp.where(hit, v, 0.0))

            sy0 = pick(y0_v)
            sx0 = pick(x0_v)
            sy1 = pick(y1_v)
            sx1 = pick(x1_v)
            sar = (sy1 - sy0) * (sx1 - sx0)
            cy0 = jnp.full((16,), sy0, jnp.float32)
            cx0 = jnp.full((16,), sx0, jnp.float32)
            cy1 = jnp.full((16,), sy1, jnp.float32)
            cx1 = jnp.full((16,), sx1, jnp.float32)
            car = jnp.full((16,), sar, jnp.float32)

            nkc = (kn + 15) // 16

            def ibody(j, mx):
                a0 = ky0_v[pl.ds(j * 16, 16)]
                b0 = kx0_v[pl.ds(j * 16, 16)]
                a1 = ky1_v[pl.ds(j * 16, 16)]
                b1 = kx1_v[pl.ds(j * 16, 16)]
                ar = kar_v[pl.ds(j * 16, 16)]
                yi0 = jnp.maximum(cy0, a0)
                xi0 = jnp.maximum(cx0, b0)
                yi1 = jnp.minimum(cy1, a1)
                xi1 = jnp.minimum(cx1, b1)
                inter = (jnp.maximum(yi1 - yi0, 0.0)
                         * jnp.maximum(xi1 - xi0, 0.0))
                iou = inter / (car + ar - inter + 1e-8)
                return jnp.maximum(mx, jnp.max(iou))

            mx = lax.fori_loop(0, nkc, ibody, jnp.float32(_NEG_INF))
            keep = live & (mx <= iou_thr)

            @pl.when(keep)
            def _append():
                _store1(ky0_v, kn, sy0)
                _store1(kx0_v, kn, sx0)
                _store1(ky1_v, kn, sy1)
                _store1(kx1_v, kn, sx1)
                _store1(kar_v, kn, sar)

            kn = kn + jnp.where(keep, jnp.int32(1), jnp.int32(0))
            return (kn, jnp.where(live, jnp.int32(1), jnp.int32(0)))

        kn, _ = lax.while_loop(wcond, wbody,
                               (jnp.int32(0), jnp.int32(1)))

        b0y0 = jnp.full((16,), y0_v[pl.ds(0, 16)][0], jnp.float32)
        b0x0 = jnp.full((16,), x0_v[pl.ds(0, 16)][0], jnp.float32)
        b0y1 = jnp.full((16,), y1_v[pl.ds(0, 16)][0], jnp.float32)
        b0x1 = jnp.full((16,), x1_v[pl.ds(0, 16)][0], jnp.float32)
        for j in range(_KCAP // 16):
            kidx = j * 16 + iota
            mask = kidx >= kn
            ky0_v[pl.ds(j * 16, 16)] = jnp.where(
                mask, b0y0, ky0_v[pl.ds(j * 16, 16)])
            kx0_v[pl.ds(j * 16, 16)] = jnp.where(
                mask, b0x0, kx0_v[pl.ds(j * 16, 16)])
            ky1_v[pl.ds(j * 16, 16)] = jnp.where(
                mask, b0y1, ky1_v[pl.ds(j * 16, 16)])
            kx1_v[pl.ds(j * 16, 16)] = jnp.where(
                mask, b0x1, kx1_v[pl.ds(j * 16, 16)])

        pltpu.sync_copy(ky0_v, out_hbm.at[samp, 0])
        pltpu.sync_copy(kx0_v, out_hbm.at[samp, 1])
        pltpu.sync_copy(ky1_v, out_hbm.at[samp, 2])
        pltpu.sync_copy(kx1_v, out_hbm.at[samp, 3])


def _sc_nms(scores, y0, x0, y1, x1, b, top_n, iou_thr, interpret=False):
    mesh = plsc.VectorSubcoreMesh(core_axis_name="c", subcore_axis_name="s",
                                  num_cores=2, num_subcores=16)
    fn = pl.kernel(
        functools.partial(_sc_nms_body, top_n, iou_thr, b),
        out_type=jax.ShapeDtypeStruct((b, 4, _KCAP), jnp.float32),
        mesh=mesh,
        scratch_types=[
            pltpu.VMEM((_A_PAD,), jnp.float32),
            pltpu.VMEM((_A_PAD,), jnp.float32),
            pltpu.VMEM((_A_PAD,), jnp.float32),
            pltpu.VMEM((_A_PAD,), jnp.float32),
            pltpu.VMEM((_A_PAD,), jnp.float32),
            pltpu.VMEM((_L1N,), jnp.float32),
            pltpu.VMEM((_L2N,), jnp.float32),
            pltpu.VMEM((_KCAP,), jnp.float32),
            pltpu.VMEM((_KCAP,), jnp.float32),
            pltpu.VMEM((_KCAP,), jnp.float32),
            pltpu.VMEM((_KCAP,), jnp.float32),
            pltpu.VMEM((_KCAP,), jnp.float32),
        ],
        compiler_params=pltpu.CompilerParams(needs_layout_passes=False),
        interpret=interpret,
    )
    return fn(scores, y0, x0, y1, x1)


def _proposal(rpn_cls, rpn_reg, anchors, top_n, iou_thr, interpret=False):
    b = rpn_cls.shape[0]
    a = rpn_cls.shape[1]
    scores, y0, x0, y1, x1 = _decode_tc(rpn_cls, rpn_reg, anchors, b, a,
                                        interpret=interpret)
    out = _sc_nms(scores, y0, x0, y1, x1, b, top_n, iou_thr,
                  interpret=interpret)
    proposals = jnp.transpose(out, (0, 2, 1))[:, :top_n, :].reshape(
        b * top_n, 4)
    indices = jnp.zeros((b * top_n,), jnp.int32)
    return proposals, indices


def kernel(rpn_cls, rpn_reg, anchors):
    return _proposal(rpn_cls, rpn_reg, anchors, _TOP_N, _IOU_THR)


# speedup vs baseline: 118.5101x; 1.1772x over previous
import functools

import jax
import jax.numpy as jnp
from jax import lax
from jax.experimental import pallas as pl
from jax.experimental.pallas import tpu as pltpu
from jax.experimental.pallas import tpu_sc as plsc

_A = 20000
_B = 8
_TOP_N = 200
_IOU_THR = 0.7
_LANES = 128

_A_PAD = 20480
_ROWS = _A_PAD // _LANES
_KCAP = 208
_L1N = _A_PAD // 16
_L2N = _L1N // 16
_BIG = 1 << 30
_NEG_INF = float("-inf")


def _decode_body(a_valid, cls_ref, reg_ref, anch_ref,
                 sc_ref, y0_ref, x0_ref, y1_ref, x1_ref):
    row_iota = lax.broadcasted_iota(jnp.int32, (_ROWS, _LANES), 0)
    col_iota = lax.broadcasted_iota(jnp.int32, (_ROWS, _LANES), 1)
    valid = (row_iota * _LANES + col_iota) < a_valid

    scores = jax.nn.sigmoid(cls_ref[0])
    sc_ref[0] = jnp.where(valid, scores, _NEG_INF)

    aymin = anch_ref[0]
    axmin = anch_ref[1]
    aymax = anch_ref[2]
    axmax = anch_ref[3]
    ah = aymax - aymin
    aw = axmax - axmin
    acy = aymin + 0.5 * ah
    acx = axmin + 0.5 * aw
    dy = reg_ref[0, 0]
    dx = reg_ref[0, 1]
    dh = reg_ref[0, 2]
    dw = reg_ref[0, 3]
    pcy = dy * ah + acy
    pcx = dx * aw + acx
    ph = jnp.exp(dh) * ah
    pw = jnp.exp(dw) * aw
    y0_ref[0] = jnp.clip(pcy - 0.5 * ph, 0.0, 1.0)
    x0_ref[0] = jnp.clip(pcx - 0.5 * pw, 0.0, 1.0)
    y1_ref[0] = jnp.clip(pcy + 0.5 * ph, 0.0, 1.0)
    x1_ref[0] = jnp.clip(pcx + 0.5 * pw, 0.0, 1.0)


def _decode_tc(rpn_cls, rpn_reg, anchors, b, a_valid, interpret=False):
    pad = _A_PAD - a_valid
    cls_p = jnp.pad(rpn_cls[..., 0], ((0, 0), (0, pad))).reshape(
        b, _ROWS, _LANES)
    reg_p = jnp.pad(jnp.transpose(rpn_reg, (0, 2, 1)),
                    ((0, 0), (0, 0), (0, pad))).reshape(b, 4, _ROWS, _LANES)
    anch_p = jnp.pad(jnp.transpose(anchors, (1, 0)),
                     ((0, 0), (0, pad))).reshape(4, _ROWS, _LANES)

    plane = jax.ShapeDtypeStruct((b, _ROWS, _LANES), jnp.float32)
    outs = pl.pallas_call(
        functools.partial(_decode_body, a_valid),
        grid=(b,),
        in_specs=[
            pl.BlockSpec((1, _ROWS, _LANES), lambda i: (i, 0, 0)),
            pl.BlockSpec((1, 4, _ROWS, _LANES), lambda i: (i, 0, 0, 0)),
            pl.BlockSpec((4, _ROWS, _LANES), lambda i: (0, 0, 0)),
        ],
        out_specs=[pl.BlockSpec((1, _ROWS, _LANES), lambda i: (i, 0, 0))] * 5,
        out_shape=[plane] * 5,
        interpret=interpret,
    )(cls_p, reg_p, anch_p)
    return [o.reshape(b, _A_PAD) for o in outs]


def _axis_ids():
    return lax.axis_index("c"), lax.axis_index("s")


def _store1(ref, pos, val):
    plsc.store_scatter(ref, [jnp.full((16,), pos, jnp.int32)],
                       jnp.full((16,), val, jnp.float32),
                       mask=lax.iota(jnp.int32, 16) == 0)


def _sc_nms_body(top_n, iou_thr, n_samples,
                 sc_hbm, y0_hbm, x0_hbm, y1_hbm, x1_hbm, out_hbm,
                 sc_v, y0_v, x0_v, y1_v, x1_v, l1_v, l2_v,
                 ky0_v, kx0_v, ky1_v, kx1_v, kar_v, sem_a, sem_b):
    c, s = _axis_ids()
    n_cores = 2
    per_core = n_samples // n_cores

    @pl.when(s < per_core)
    def _work():
        samp = c * per_core + s
        h_sc = pltpu.async_copy(sc_hbm.at[samp], sc_v, sem_a)
        h_y0 = pltpu.async_copy(y0_hbm.at[samp], y0_v, sem_b)
        h_x0 = pltpu.async_copy(x0_hbm.at[samp], x0_v, sem_b)
        h_y1 = pltpu.async_copy(y1_hbm.at[samp], y1_v, sem_b)
        h_x1 = pltpu.async_copy(x1_hbm.at[samp], x1_v, sem_b)

        iota = lax.iota(jnp.int32, 16)
        zeros16 = jnp.zeros((16,), jnp.float32)

        for j in range(_KCAP // 16):
            ky0_v[pl.ds(j * 16, 16)] = zeros16
            kx0_v[pl.ds(j * 16, 16)] = zeros16
            ky1_v[pl.ds(j * 16, 16)] = zeros16
            kx1_v[pl.ds(j * 16, 16)] = zeros16
            kar_v[pl.ds(j * 16, 16)] = zeros16

        h_sc.wait()

        def l1_build(i, _):
            for u in range(4):
                ch = sc_v[pl.ds((i * 4 + u) * 16, 16)]
                _store1(l1_v, i * 4 + u, jnp.max(ch))
            return 0
        lax.fori_loop(0, _L1N // 4, l1_build, 0)

        def l2_build(i, _):
            for u in range(4):
                ch = l1_v[pl.ds((i * 4 + u) * 16, 16)]
                _store1(l2_v, i * 4 + u, jnp.max(ch))
            return 0
        lax.fori_loop(0, _L2N // 4, l2_build, 0)

        h_y0.wait()
        h_x0.wait()
        h_y1.wait()
        h_x1.wait()

        def wcond(state):
            kn, alive = state
            return (kn < top_n) & (alive > 0)

        def wbody(state):
            kn, alive = state

            l2chunks = [l2_v[pl.ds(k * 16, 16)] for k in range(_L2N // 16)]
            vmax = l2chunks[0]
            for ch in l2chunks[1:]:
                vmax = jnp.maximum(vmax, ch)
            m = jnp.max(vmax)

            cand = jnp.where(l2chunks[0] == m, iota, _BIG)
            for k, ch in enumerate(l2chunks[1:]):
                cand = jnp.minimum(
                    cand, jnp.where(ch == m, (k + 1) * 16 + iota, _BIG))
            p2 = jnp.min(cand)

            ch1 = l1_v[pl.ds(p2 * 16, 16)]
            f1 = plsc.all_reduce_ffs(ch1 == m)
            p1 = p2 * 16 + f1[0]

            ch0 = sc_v[pl.ds(p1 * 16, 16)]
            f0 = plsc.all_reduce_ffs(ch0 == m)
            hit0 = iota == f0

            live = m > _NEG_INF

            @pl.when(live)
            def _consume():
                ch0n = jnp.where(hit0, _NEG_INF, ch0)
                sc_v[pl.ds(p1 * 16, 16)] = ch0n
                m1 = jnp.max(ch0n)
                _store1(l1_v, p1, m1)
                ch1n = jnp.where(iota == f1, m1, ch1)
                _store1(l2_v, p2, jnp.max(ch1n))

            pvec = p1 * 16 + f0
            cy0 = plsc.load_gather(y0_v, [pvec])
            cx0 = plsc.load_gather(x0_v, [pvec])
            cy1 = plsc.load_gather(y1_v, [pvec])
            cx1 = plsc.load_gather(x1_v, [pvec])
            car = (cy1 - cy0) * (cx1 - cx0)

            nkc = (kn + 15) // 16

            def ibody(j, mxv):
                a0 = ky0_v[pl.ds(j * 16, 16)]
                b0 = kx0_v[pl.ds(j * 16, 16)]
                a1 = ky1_v[pl.ds(j * 16, 16)]
                b1 = kx1_v[pl.ds(j * 16, 16)]
                ar = kar_v[pl.ds(j * 16, 16)]
                yi0 = jnp.maximum(cy0, a0)
                xi0 = jnp.maximum(cx0, b0)
                yi1 = jnp.minimum(cy1, a1)
                xi1 = jnp.minimum(cx1, b1)
                inter = (jnp.maximum(yi1 - yi0, 0.0)
                         * jnp.maximum(xi1 - xi0, 0.0))
                iou = inter / (car + ar - inter + 1e-8)
                return jnp.maximum(mxv, iou)

            mxv = lax.fori_loop(0, nkc, ibody,
                                jnp.full((16,), _NEG_INF, jnp.float32))
            keep = live & (jnp.max(mxv) <= iou_thr)

            @pl.when(keep)
            def _append():
                _store1(ky0_v, kn, cy0[0])
                _store1(kx0_v, kn, cx0[0])
                _store1(ky1_v, kn, cy1[0])
                _store1(kx1_v, kn, cx1[0])
                _store1(kar_v, kn, car[0])

            kn = kn + jnp.where(keep, jnp.int32(1), jnp.int32(0))
            return (kn, jnp.where(live, jnp.int32(1), jnp.int32(0)))

        kn, _ = lax.while_loop(wcond, wbody,
                               (jnp.int32(0), jnp.int32(1)))

        b0y0 = jnp.full((16,), y0_v[pl.ds(0, 16)][0], jnp.float32)
        b0x0 = jnp.full((16,), x0_v[pl.ds(0, 16)][0], jnp.float32)
        b0y1 = jnp.full((16,), y1_v[pl.ds(0, 16)][0], jnp.float32)
        b0x1 = jnp.full((16,), x1_v[pl.ds(0, 16)][0], jnp.float32)
        for j in range(_KCAP // 16):
            kidx = j * 16 + iota
            mask = kidx >= kn
            ky0_v[pl.ds(j * 16, 16)] = jnp.where(
                mask, b0y0, ky0_v[pl.ds(j * 16, 16)])
            kx0_v[pl.ds(j * 16, 16)] = jnp.where(
                mask, b0x0, kx0_v[pl.ds(j * 16, 16)])
            ky1_v[pl.ds(j * 16, 16)] = jnp.where(
                mask, b0y1, ky1_v[pl.ds(j * 16, 16)])
            kx1_v[pl.ds(j * 16, 16)] = jnp.where(
                mask, b0x1, kx1_v[pl.ds(j * 16, 16)])

        pltpu.sync_copy(ky0_v, out_hbm.at[samp, 0])
        pltpu.sync_copy(kx0_v, out_hbm.at[samp, 1])
        pltpu.sync_copy(ky1_v, out_hbm.at[samp, 2])
        pltpu.sync_copy(kx1_v, out_hbm.at[samp, 3])


def _sc_nms(scores, y0, x0, y1, x1, b, top_n, iou_thr, interpret=False):
    mesh = plsc.VectorSubcoreMesh(core_axis_name="c", subcore_axis_name="s",
                                  num_cores=2, num_subcores=16)
    fn = pl.kernel(
        functools.partial(_sc_nms_body, top_n, iou_thr, b),
        out_type=jax.ShapeDtypeStruct((b, 4, _KCAP), jnp.float32),
        mesh=mesh,
        scratch_types=[
            pltpu.VMEM((_A_PAD,), jnp.float32),
            pltpu.VMEM((_A_PAD,), jnp.float32),
            pltpu.VMEM((_A_PAD,), jnp.float32),
            pltpu.VMEM((_A_PAD,), jnp.float32),
            pltpu.VMEM((_A_PAD,), jnp.float32),
            pltpu.VMEM((_L1N,), jnp.float32),
            pltpu.VMEM((_L2N,), jnp.float32),
            pltpu.VMEM((_KCAP,), jnp.float32),
            pltpu.VMEM((_KCAP,), jnp.float32),
            pltpu.VMEM((_KCAP,), jnp.float32),
            pltpu.VMEM((_KCAP,), jnp.float32),
            pltpu.VMEM((_KCAP,), jnp.float32),
            pltpu.SemaphoreType.DMA,
            pltpu.SemaphoreType.DMA,
        ],
        compiler_params=pltpu.CompilerParams(needs_layout_passes=False),
        interpret=interpret,
    )
    return fn(scores, y0, x0, y1, x1)


def _proposal(rpn_cls, rpn_reg, anchors, top_n, iou_thr, interpret=False):
    b = rpn_cls.shape[0]
    a = rpn_cls.shape[1]
    scores, y0, x0, y1, x1 = _decode_tc(rpn_cls, rpn_reg, anchors, b, a,
                                        interpret=interpret)
    out = _sc_nms(scores, y0, x0, y1, x1, b, top_n, iou_thr,
                  interpret=interpret)
    proposals = jnp.transpose(out, (0, 2, 1))[:, :top_n, :].reshape(
        b * top_n, 4)
    indices = jnp.zeros((b * top_n,), jnp.int32)
    return proposals, indices


def kernel(rpn_cls, rpn_reg, anchors):
    return _proposal(rpn_cls, rpn_reg, anchors, _TOP_N, _IOU_THR)
